# baseline (speedup 1.0000x reference)
import functools

import jax
import jax.numpy as jnp
from jax import lax
from jax.experimental import pallas as pl
from jax.experimental.pallas import tpu as pltpu
from jax.experimental.pallas import tpu_sc as plsc

NUM_TICKERS = 1000000
DIM = 32
BATCH = 16384
LENGTH = 50

_NUM_CORES = 2
_NUM_SUBCORES = 16
_NW = _NUM_CORES * _NUM_SUBCORES
_B_PER_W = BATCH // _NW
_CHUNK = 128
_N_CHUNK = _B_PER_W // _CHUNK

_sc_mesh = plsc.VectorSubcoreMesh(core_axis_name="c", subcore_axis_name="s")


@functools.partial(
    pl.kernel,
    out_type=jax.ShapeDtypeStruct((BATCH, DIM), jnp.float32),
    mesh=_sc_mesh,
    scratch_types=[
        pltpu.VMEM((_N_CHUNK, _CHUNK), jnp.int32),
        pltpu.VMEM((_B_PER_W, DIM), jnp.float32),
        pltpu.SemaphoreType.DMA,
    ],
    compiler_params=pltpu.CompilerParams(use_tc_tiling_on_sc=False),
)
def _sc_gather(table_hbm, idx_hbm, out_hbm, idx_v, rows_v, sem):
    wid = lax.axis_index("s") * _NUM_CORES + lax.axis_index("c")
    base = wid * _B_PER_W
    pltpu.sync_copy(idx_hbm.at[wid], idx_v)
    copies = []
    for j in range(_N_CHUNK):
        copies.append(
            pltpu.make_async_copy(
                table_hbm.at[idx_v.at[j]],
                rows_v.at[pl.ds(j * _CHUNK, _CHUNK)],
                sem,
            )
        )
    for c in copies:
        c.start()
    for c in copies:
        c.wait()
    pltpu.sync_copy(rows_v, out_hbm.at[pl.ds(base, _B_PER_W)])


_BS = 256


def _expand_body(delta_ref, emb_ref, out_ref):
    delta = delta_ref[0, 0]
    rows = emb_ref[...][:, None, :] + delta
    out_ref[...] = jnp.broadcast_to(rows, (_BS, LENGTH, DIM))


@jax.jit
def _tc_expand(delta, emb):
    return pl.pallas_call(
        _expand_body,
        grid=(BATCH // _BS,),
        in_specs=[
            pl.BlockSpec(memory_space=pltpu.SMEM),
            pl.BlockSpec((_BS, DIM), lambda i: (i, 0)),
        ],
        out_specs=pl.BlockSpec((_BS, LENGTH, DIM), lambda i: (i, 0, 0)),
        out_shape=jax.ShapeDtypeStruct((BATCH, LENGTH, DIM), jnp.float32),
    )(delta, emb)


def kernel(ticker_ids, length, table):
    ids = ticker_ids.astype(jnp.int32).reshape(_NW, _N_CHUNK, _CHUNK)
    emb = _sc_gather(table, ids)
    delta = (jnp.asarray(length, jnp.float32) - LENGTH).reshape(1, 1)
    return _tc_expand(delta, emb)
